# baseline (speedup 1.0000x reference)
import functools
import jax
import jax.numpy as jnp
from jax import lax
from jax.experimental import pallas as pl
from jax.experimental.pallas import tpu as pltpu
from jax.experimental.pallas import tpu_sc as plsc

N = 4096
CUTOFF_SQ = 0.25 * 0.25
NW = 32
ROWS_PER_W = N // NW
L = 16


def _sc_kernel(x_hbm, y_hbm, z_hbm, xr_hbm, yr_hbm, zr_hbm, wb_hbm, out_hbm,
               xv, yv, zv, wbv, row_v, xrv, yrv, zrv):
    wid = lax.axis_index("s") * 2 + lax.axis_index("c")
    base = wid * ROWS_PER_W
    pltpu.sync_copy(x_hbm, xv)
    pltpu.sync_copy(y_hbm, yv)
    pltpu.sync_copy(z_hbm, zv)
    pltpu.sync_copy(wb_hbm, wbv)
    pltpu.sync_copy(xr_hbm.at[pl.ds(base * L, ROWS_PER_W * L)], xrv)
    pltpu.sync_copy(yr_hbm.at[pl.ds(base * L, ROWS_PER_W * L)], yrv)
    pltpu.sync_copy(zr_hbm.at[pl.ds(base * L, ROWS_PER_W * L)], zrv)
    w16 = wbv[pl.ds(0, L)]
    b16 = wbv[pl.ds(L, L)]
    half = jnp.full((L,), 0.5, jnp.float32)
    three_half = jnp.full((L,), 1.5, jnp.float32)
    one = jnp.full((L,), 1.0, jnp.float32)
    zero = jnp.zeros((L,), jnp.float32)
    c2 = jnp.full((L,), CUTOFF_SQ, jnp.float32)
    magic = jnp.full((L,), 0x5F3759DF, jnp.int32)

    def row_body(r, carry):
        i = base + r
        xi = xrv[pl.ds(r * L, L)]
        yi = yrv[pl.ds(r * L, L)]
        zi = zrv[pl.ds(r * L, L)]

        def col_body(c, carry2):
            c0 = c * L
            dsq = zero
            for src, ci in ((xv, xi), (yv, yi), (zv, zi)):
                d = src[pl.ds(c0, L)] - ci
                a = jnp.abs(d)
                m = jnp.minimum(a, one - a)
                dsq = dsq + m * m
            mask = (dsq < c2) & (dsq != zero)
            yk = lax.bitcast_convert_type(
                magic - lax.shift_right_logical(
                    lax.bitcast_convert_type(dsq, jnp.int32), 1),
                jnp.float32)
            hx = half * dsq
            for _ in range(3):
                yk = yk * (three_half - hx * yk * yk)
            rr = dsq * yk
            row_v[pl.ds(c0, L)] = jnp.where(mask, rr * w16 + b16, zero)
            return carry2

        lax.fori_loop(0, N // L, col_body, 0, unroll=4)
        pltpu.sync_copy(row_v, out_hbm.at[pl.ds(i * N, N)])
        return carry

    lax.fori_loop(0, ROWS_PER_W, row_body, 0)


def kernel(xyz, W, b):
    xs = xyz[:, 0]
    ys = xyz[:, 1]
    zs = xyz[:, 2]
    wb = jnp.concatenate([
        jnp.broadcast_to(W.reshape(()), (L,)),
        jnp.broadcast_to(b.reshape(()), (L,)),
    ])
    mesh = plsc.VectorSubcoreMesh(core_axis_name="c", subcore_axis_name="s")
    out = functools.partial(
        pl.kernel,
        mesh=mesh,
        out_type=jax.ShapeDtypeStruct((N * N,), jnp.float32),
        scratch_types=[
            pltpu.VMEM((N,), jnp.float32),
            pltpu.VMEM((N,), jnp.float32),
            pltpu.VMEM((N,), jnp.float32),
            pltpu.VMEM((2 * L,), jnp.float32),
            pltpu.VMEM((N,), jnp.float32),
            pltpu.VMEM((ROWS_PER_W * L,), jnp.float32),
            pltpu.VMEM((ROWS_PER_W * L,), jnp.float32),
            pltpu.VMEM((ROWS_PER_W * L,), jnp.float32),
        ],
    )(_sc_kernel)(xs, ys, zs, jnp.repeat(xs, L), jnp.repeat(ys, L),
                  jnp.repeat(zs, L), wb)
    return jax.lax.reshape(out, (N, N, 1))


# speedup vs baseline: 16.7872x; 16.7872x over previous
import jax
import jax.numpy as jnp
from jax import lax
from jax.experimental import pallas as pl

N = 4096
CUTOFF_SQ = 0.25 * 0.25
BR = 512


def _pair_kernel(xi_ref, xt4_ref, wb_ref, out_ref):
    w = wb_ref[0, 0]
    b = wb_ref[0, 1]
    dsq = jnp.zeros((BR, 4, 8, 128), jnp.float32)
    for k in range(3):
        xj = xt4_ref[k:k + 1]
        xi = xi_ref[:, k:k + 1][..., None, None]
        d = xj - xi
        d = d - jnp.round(d)
        dsq = dsq + d * d
    mask = (dsq < CUTOFF_SQ) & (dsq != 0.0)
    r = dsq * lax.rsqrt(dsq)
    out_ref[...] = jnp.where(mask, r * w + b, 0.0)


def kernel(xyz, W, b):
    xt4 = xyz.T.reshape(3, 4, 8, 128)
    wb = jnp.concatenate([W.reshape(1, 1), b.reshape(1, 1)], axis=1)
    out = pl.pallas_call(
        _pair_kernel,
        grid=(N // BR,),
        in_specs=[
            pl.BlockSpec((BR, 3), lambda i: (i, 0)),
            pl.BlockSpec((3, 4, 8, 128), lambda i: (0, 0, 0, 0)),
            pl.BlockSpec((1, 2), lambda i: (0, 0)),
        ],
        out_specs=pl.BlockSpec((BR, 4, 8, 128), lambda i: (i, 0, 0, 0)),
        out_shape=jax.ShapeDtypeStruct((N, 4, 8, 128), jnp.float32),
    )(xyz, xt4, wb)
    return jax.lax.reshape(out, (N, N, 1))


# speedup vs baseline: 17.0779x; 1.0173x over previous
import jax
import jax.numpy as jnp
from jax import lax
from jax.experimental import pallas as pl

N = 4096
CUTOFF_SQ = 0.25 * 0.25
BR = 256


def _pair_kernel(xi_ref, xt4_ref, wb_ref, out_ref):
    w = wb_ref[0, 0]
    b = wb_ref[0, 1]
    dsq = None
    for k in range(3):
        xj = xt4_ref[k:k + 1]
        xi = xi_ref[:, k:k + 1][..., None, None]
        d = xj - xi
        d = d - jnp.round(d)
        dsq = d * d if dsq is None else dsq + d * d
    mask = (dsq < CUTOFF_SQ) & (dsq != 0.0)
    r = dsq * lax.rsqrt(dsq)
    out_ref[...] = jnp.where(mask, r * w + b, 0.0)


def kernel(xyz, W, b):
    xt4 = xyz.T.reshape(3, 4, 8, 128)
    wb = jnp.concatenate([W.reshape(1, 1), b.reshape(1, 1)], axis=1)
    out = pl.pallas_call(
        _pair_kernel,
        grid=(N // BR,),
        in_specs=[
            pl.BlockSpec((BR, 3), lambda i: (i, 0)),
            pl.BlockSpec((3, 4, 8, 128), lambda i: (0, 0, 0, 0)),
            pl.BlockSpec((1, 2), lambda i: (0, 0)),
        ],
        out_specs=pl.BlockSpec((BR, 4, 8, 128), lambda i: (i, 0, 0, 0)),
        out_shape=jax.ShapeDtypeStruct((N, 4, 8, 128), jnp.float32),
    )(xyz, xt4, wb)
    return jax.lax.reshape(out, (N, N, 1))
